# Initial kernel scaffold; baseline (speedup 1.0000x reference)
import functools

import jax
import jax.numpy as jnp
from jax import lax
from jax.experimental import pallas as pl
from jax.experimental.pallas import tpu as pltpu
from jax.experimental.pallas import tpu_sc as plsc

_N = 10000
_E = 320000
_S = 2
_D = 128

_NS = 16
_C = 128
_K = 157
_EPT = _K * _C
_EP = _NS * _EPT
_RPT = _N // _NS


def _matmul_body(x_ref, w_ref, o_ref):
    o_ref[0] = jnp.dot(x_ref[...], w_ref[0], preferred_element_type=jnp.float32)


def _matmul(x, W):
    return pl.pallas_call(
        _matmul_body,
        grid=(_S, 10),
        in_specs=[
            pl.BlockSpec((_N // 10, _D), lambda s, j: (j, 0)),
            pl.BlockSpec((1, _D, _D), lambda s, j: (s, 0, 0)),
        ],
        out_specs=pl.BlockSpec((1, _N // 10, _D), lambda s, j: (s, j, 0)),
        out_shape=jax.ShapeDtypeStruct((_S, _N, _D), jnp.float32),
    )(x, W)


def _combine_body(p_ref, b_ref, o_ref):
    o_ref[...] = jax.nn.relu(p_ref[0] + p_ref[1] + b_ref[...])


def _combine(partial, b2d):
    return pl.pallas_call(
        _combine_body,
        grid=(10,),
        in_specs=[
            pl.BlockSpec((_S, _N // 10, _D), lambda j: (0, j, 0)),
            pl.BlockSpec((1, _D), lambda j: (0, 0)),
        ],
        out_specs=pl.BlockSpec((_N // 10, _D), lambda j: (j, 0)),
        out_shape=jax.ShapeDtypeStruct((_N, _D), jnp.float32),
    )(partial, b2d)


def _spmm_body(presup_hbm, src_hbm, dst_hbm, ev_hbm, out_hbm,
               acc, src_v, dst_v, ev_v, rows_v, sem):
    c = lax.axis_index("c")
    t = lax.axis_index("s")

    pltpu.sync_copy(src_hbm.at[c, t], src_v)
    pltpu.sync_copy(dst_hbm.at[c, t], dst_v)
    pltpu.sync_copy(ev_hbm.at[c, t], ev_v)

    def _zero_row(r, _):
        for j in range(_D // 16):
            rows_v[r, pl.ds(j * 16, 16)] = jnp.zeros((16,), jnp.float32)
        return 0
    lax.fori_loop(0, _C, _zero_row, 0)
    for jj in range(5):
        pltpu.sync_copy(rows_v.at[pl.ds(0, 125)],
                        acc.at[pl.ds(t * _RPT + jj * 125, 125)])
    plsc.subcore_barrier()

    def _chunk(k, _):
        pltpu.async_copy(presup_hbm.at[src_v.at[k]], rows_v, sem).wait()
        k16 = jnp.full((16,), k, jnp.int32)

        def _edge(e, _):
            evs = plsc.load_gather(ev_v, [k16, jnp.full((16,), e, jnp.int32)])
            for j in range(_D // 16):
                sl = pl.ds(j * 16, 16)
                rows_v[e, sl] = rows_v[e, sl] * evs
            return 0
        lax.fori_loop(0, _C, _edge, 0)

        pltpu.sync_copy(rows_v, acc.at[dst_v.at[k]], add=True)
        return 0
    lax.fori_loop(0, _K, _chunk, 0)

    plsc.subcore_barrier()
    for jj in range(5):
        r0 = t * _RPT + jj * 125
        pltpu.sync_copy(acc.at[pl.ds(r0, 125)], rows_v.at[pl.ds(0, 125)])
        pltpu.sync_copy(rows_v.at[pl.ds(0, 125)], out_hbm.at[c, pl.ds(r0, 125)])


_spmm = functools.partial(
    pl.kernel,
    out_type=jax.ShapeDtypeStruct((_S, _N, _D), jnp.float32),
    mesh=plsc.VectorSubcoreMesh(core_axis_name="c", subcore_axis_name="s",
                                num_cores=_S, num_subcores=_NS),
    scratch_types=[
        pltpu.VMEM_SHARED((_N, _D), jnp.float32),
        pltpu.VMEM((_K, _C), jnp.int32),
        pltpu.VMEM((_K, _C), jnp.int32),
        pltpu.VMEM((_K, _C), jnp.float32),
        pltpu.VMEM((_C, _D), jnp.float32),
        pltpu.SemaphoreType.DMA,
    ],
)(_spmm_body)


def kernel(x, edge_index, edge_values, W, b):
    x = x.astype(jnp.float32)
    ei = edge_index.astype(jnp.int32)
    ev = edge_values.astype(jnp.float32)
    W = W.astype(jnp.float32)

    presup = _matmul(x, W).reshape(_S * _N, _D)

    pad = _EP - _E
    dst = jnp.pad(ei[:, 0, :], ((0, 0), (0, pad))).reshape(_S, _NS, _K, _C)
    src = jnp.pad(ei[:, 1, :], ((0, 0), (0, pad)))
    src = (src + (jnp.arange(_S, dtype=jnp.int32) * _N)[:, None])
    src = src.reshape(_S, _NS, _K, _C)
    evp = jnp.pad(ev, ((0, 0), (0, pad))).reshape(_S, _NS, _K, _C)

    partial = _spmm(presup, src, dst, evp)
    return _combine(partial, b.reshape(1, _D))


# speedup vs baseline: 4.0260x; 4.0260x over previous
import functools

import jax
import jax.numpy as jnp
from jax import lax
from jax.experimental import pallas as pl
from jax.experimental.pallas import tpu as pltpu
from jax.experimental.pallas import tpu_sc as plsc

_N = 10000
_E = 320000
_S = 2
_D = 128

_NS = 16
_C = 128
_K = 160
_B = 8
_NBLK = _K // _B
_EPT = _K * _C
_EP = _NS * _EPT
_NP = 10240
_RPT = _NP // _NS


def _matmul_body(x_ref, w_ref, o_ref):
    o_ref[0] = jnp.dot(x_ref[...], w_ref[0], preferred_element_type=jnp.float32)


def _matmul(x, W):
    return pl.pallas_call(
        _matmul_body,
        grid=(_S, 10),
        in_specs=[
            pl.BlockSpec((_N // 10, _D), lambda s, j: (j, 0)),
            pl.BlockSpec((1, _D, _D), lambda s, j: (s, 0, 0)),
        ],
        out_specs=pl.BlockSpec((1, _N // 10, _D), lambda s, j: (s, j, 0)),
        out_shape=jax.ShapeDtypeStruct((_S, _N, _D), jnp.float32),
    )(x, W)


def _combine_body(p_ref, b_ref, o_ref):
    o_ref[...] = jax.nn.relu(p_ref[0] + p_ref[1] + b_ref[...])


def _combine(partial, b2d):
    return pl.pallas_call(
        _combine_body,
        grid=(10,),
        in_specs=[
            pl.BlockSpec((_S, _N // 10, _D), lambda j: (0, j, 0)),
            pl.BlockSpec((1, _D), lambda j: (0, 0)),
        ],
        out_specs=pl.BlockSpec((_N // 10, _D), lambda j: (j, 0)),
        out_shape=jax.ShapeDtypeStruct((_N, _D), jnp.float32),
    )(partial, b2d)


def _spmm_body(presup_hbm, src_hbm, dst_hbm, ev_hbm, out_hbm,
               acc, src_v, dst_v, ev_v, rows_v, sem):
    c = lax.axis_index("c")
    t = lax.axis_index("s")

    def _zero_row(r, _):
        for j in range(_D // 16):
            rows_v[r, pl.ds(j * 16, 16)] = jnp.zeros((16,), jnp.float32)
        return 0
    lax.fori_loop(0, _C, _zero_row, 0)
    for jj in range(5):
        pltpu.sync_copy(rows_v, acc.at[pl.ds(t * _RPT + jj * _C, _C)])
    plsc.subcore_barrier()

    def _block(bb, _):
        sl_b = pl.ds(bb * _B, _B)
        pltpu.sync_copy(src_hbm.at[c, t, sl_b], src_v)
        pltpu.sync_copy(dst_hbm.at[c, t, sl_b], dst_v)
        pltpu.sync_copy(ev_hbm.at[c, t, sl_b], ev_v)

        def _chunk(k, _):
            pltpu.async_copy(presup_hbm.at[src_v.at[k]], rows_v, sem).wait()

            def _group(g, _):
                evg = ev_v[k, pl.ds(g * 16, 16)]
                for l in range(16):
                    e = g * 16 + l
                    evs = jnp.full((16,), evg[l], jnp.float32)
                    for j in range(_D // 16):
                        sl = pl.ds(j * 16, 16)
                        rows_v[e, sl] = rows_v[e, sl] * evs
                return 0
            lax.fori_loop(0, _C // 16, _group, 0)

            pltpu.sync_copy(rows_v, acc.at[dst_v.at[k]], add=True)
            return 0
        lax.fori_loop(0, _B, _chunk, 0)
        return 0
    lax.fori_loop(0, _NBLK, _block, 0)

    plsc.subcore_barrier()
    for jj in range(5):
        r0 = t * _RPT + jj * _C
        pltpu.sync_copy(acc.at[pl.ds(r0, _C)], rows_v)
        pltpu.sync_copy(rows_v, out_hbm.at[c, pl.ds(r0, _C)])


_spmm = functools.partial(
    pl.kernel,
    out_type=jax.ShapeDtypeStruct((_S, _NP, _D), jnp.float32),
    mesh=plsc.VectorSubcoreMesh(core_axis_name="c", subcore_axis_name="s",
                                num_cores=_S, num_subcores=_NS),
    scratch_types=[
        pltpu.VMEM_SHARED((_NP, _D), jnp.float32),
        pltpu.VMEM((_B, _C), jnp.int32),
        pltpu.VMEM((_B, _C), jnp.int32),
        pltpu.VMEM((_B, _C), jnp.float32),
        pltpu.VMEM((_C, _D), jnp.float32),
        pltpu.SemaphoreType.DMA,
    ],
)(_spmm_body)


def kernel(x, edge_index, edge_values, W, b):
    x = x.astype(jnp.float32)
    ei = edge_index.astype(jnp.int32)
    ev = edge_values.astype(jnp.float32)
    W = W.astype(jnp.float32)

    presup = _matmul(x, W).reshape(_S * _N, _D)

    pad = _EP - _E
    dst = jnp.pad(ei[:, 0, :], ((0, 0), (0, pad))).reshape(_S, _NS, _K, _C)
    src = jnp.pad(ei[:, 1, :], ((0, 0), (0, pad)))
    src = (src + (jnp.arange(_S, dtype=jnp.int32) * _N)[:, None])
    src = src.reshape(_S, _NS, _K, _C)
    evp = jnp.pad(ev, ((0, 0), (0, pad))).reshape(_S, _NS, _K, _C)
    del pad

    partial = _spmm(presup, src, dst, evp)
    return _combine(partial, b.reshape(1, _D))
